# Initial kernel scaffold; baseline (speedup 1.0000x reference)
import functools

import jax
import jax.numpy as jnp
from jax import lax
from jax.experimental import pallas as pl
from jax.experimental.pallas import tpu as pltpu
from jax.experimental.pallas import tpu_sc as plsc

_N = 10000
_E = 320000
_F = 128
_G = 64

_NC = 2
_NS = 16
_NW = _NC * _NS
_K = 128
_C = 79
_EPT = _C * _K
_EPAD = _NW * _EPT
_TBL = 10016
_RPT = _TBL // _NS

_mesh = plsc.VectorSubcoreMesh(core_axis_name="c", subcore_axis_name="s")


@functools.partial(
    pl.kernel,
    out_type=jax.ShapeDtypeStruct((_NC, _TBL, 16), jnp.float32),
    mesh=_mesh,
    scratch_types=[
        pltpu.VMEM((_C, _K), jnp.int32),
        pltpu.VMEM((_K, 16), jnp.float32),
        pltpu.VMEM_SHARED((_TBL, 16), jnp.float32),
    ],
)
def _sc_deg(dst_hbm, ones_hbm, zeros_hbm, out_hbm, idx_v, ones_v, deg_sp):
    c = lax.axis_index("c")
    s = lax.axis_index("s")
    wid = c * _NS + s
    pltpu.sync_copy(dst_hbm.at[pl.ds(wid * _C, _C)], idx_v)
    pltpu.sync_copy(ones_hbm, ones_v)
    pltpu.sync_copy(zeros_hbm.at[pl.ds(s * _RPT, _RPT)],
                    deg_sp.at[pl.ds(s * _RPT, _RPT)])
    plsc.subcore_barrier()

    def body(j, carry):
        pltpu.sync_copy(ones_v, deg_sp.at[idx_v.at[j]], add=True)
        return carry

    lax.fori_loop(0, _C, body, 0)
    plsc.subcore_barrier()
    pltpu.sync_copy(deg_sp.at[pl.ds(s * _RPT, _RPT)],
                    out_hbm.at[c, pl.ds(s * _RPT, _RPT)])


@functools.partial(
    pl.kernel,
    out_type=jax.ShapeDtypeStruct((_NC, _TBL, _F), jnp.float32),
    mesh=_mesh,
    scratch_types=[
        pltpu.VMEM((_C, _K), jnp.int32),
        pltpu.VMEM((_C, _K), jnp.int32),
        pltpu.VMEM((_K, _F), jnp.float32),
        pltpu.VMEM_SHARED((_TBL, _F), jnp.float32),
        pltpu.SemaphoreType.DMA,
    ],
)
def _sc_scatter(src_hbm, dst_hbm, hs_hbm, zeros_hbm, out_hbm,
                isrc_v, idst_v, rows_v, acc_sp, sem):
    c = lax.axis_index("c")
    s = lax.axis_index("s")
    wid = c * _NS + s
    pltpu.sync_copy(src_hbm.at[pl.ds(wid * _C, _C)], isrc_v)
    pltpu.sync_copy(dst_hbm.at[pl.ds(wid * _C, _C)], idst_v)
    pltpu.sync_copy(zeros_hbm.at[pl.ds(s * _RPT, _RPT)],
                    acc_sp.at[pl.ds(s * _RPT, _RPT)])
    plsc.subcore_barrier()

    def body(j, carry):
        pltpu.async_copy(hs_hbm.at[isrc_v.at[j]], rows_v, sem).wait()
        pltpu.sync_copy(rows_v, acc_sp.at[idst_v.at[j]], add=True)
        return carry

    lax.fori_loop(0, _C, body, 0)
    plsc.subcore_barrier()
    pltpu.sync_copy(acc_sp.at[pl.ds(s * _RPT, _RPT)],
                    out_hbm.at[c, pl.ds(s * _RPT, _RPT)])


def _dinv_body(degs_ref, out_ref):
    d = degs_ref[0, :, 0:1] + degs_ref[1, :, 0:1] + 1.0
    out_ref[...] = jnp.broadcast_to(lax.rsqrt(d), (_TBL, _F))


def _tc_dinv(degs):
    return pl.pallas_call(
        _dinv_body,
        out_shape=jax.ShapeDtypeStruct((_TBL, _F), jnp.float32),
    )(degs)


_BR = 1000


def _mm_body(x_ref, dinv_ref, w_ref, out_ref):
    out_ref[...] = dinv_ref[...] * jnp.dot(
        x_ref[...], w_ref[...], preferred_element_type=jnp.float32)


def _tc_mm(x, dinv, w):
    return pl.pallas_call(
        _mm_body,
        grid=(_N // _BR,),
        in_specs=[
            pl.BlockSpec((_BR, _F), lambda i: (i, 0)),
            pl.BlockSpec((_BR, _F), lambda i: (i, 0)),
            pl.BlockSpec((_F, _F), lambda i: (0, 0)),
        ],
        out_specs=pl.BlockSpec((_BR, _F), lambda i: (i, 0)),
        out_shape=jax.ShapeDtypeStruct((_N, _F), jnp.float32),
    )(x, dinv, w)


def _mid_body(a0_ref, a1_ref, hs_ref, dinv_ref, b_ref, w_ref, out_ref):
    h = a0_ref[...] + a1_ref[...] + hs_ref[...]
    h = jnp.maximum(dinv_ref[...] * h + b_ref[...], 0.0)
    out_ref[...] = dinv_ref[...] * jnp.dot(
        h, w_ref[...], preferred_element_type=jnp.float32)


def _tc_mid(a0, a1, hs, dinv, b, w):
    blk = pl.BlockSpec((_BR, _F), lambda i: (i, 0))
    return pl.pallas_call(
        _mid_body,
        grid=(_N // _BR,),
        in_specs=[blk, blk, blk, blk,
                  pl.BlockSpec((1, _F), lambda i: (0, 0)),
                  pl.BlockSpec((_F, _F), lambda i: (0, 0))],
        out_specs=blk,
        out_shape=jax.ShapeDtypeStruct((_N, _F), jnp.float32),
    )(a0, a1, hs, dinv, b, w)


def _final_body(a0_ref, a1_ref, hs_ref, dinv_ref, b_ref, batch_ref,
                wh1_ref, bh1_ref, wh2_ref, bh2_ref, out_ref,
                sums_s, cnt_s):
    i = pl.program_id(0)

    @pl.when(i == 0)
    def _():
        sums_s[...] = jnp.zeros_like(sums_s)
        cnt_s[...] = jnp.zeros_like(cnt_s)

    h = a0_ref[...] + a1_ref[...] + hs_ref[...]
    h = jnp.maximum(dinv_ref[...] * h + b_ref[...], 0.0)
    b_row = batch_ref[0]
    gid = lax.broadcasted_iota(jnp.int32, (_G, _BR), 0)
    onehot = (gid == b_row).astype(jnp.float32)
    sums_s[...] += jnp.dot(onehot, h, preferred_element_type=jnp.float32)
    cnt = jnp.sum(onehot, axis=1, keepdims=True)
    cnt_s[...] += jnp.broadcast_to(cnt, (_G, _F))

    @pl.when(i == _N // _BR - 1)
    def _():
        g = sums_s[...] / jnp.maximum(cnt_s[...], 1.0)
        z = jnp.maximum(
            jnp.dot(g, wh1_ref[...], preferred_element_type=jnp.float32)
            + bh1_ref[...], 0.0)
        out_ref[...] = jnp.dot(
            z, wh2_ref[...], preferred_element_type=jnp.float32) + bh2_ref[...]


def _tc_final(a0, a1, hs, dinv, b, batch3d, wh1, bh1, wh2, bh2):
    blk = pl.BlockSpec((_BR, _F), lambda i: (i, 0))
    return pl.pallas_call(
        _final_body,
        grid=(_N // _BR,),
        in_specs=[blk, blk, blk, blk,
                  pl.BlockSpec((1, _F), lambda i: (0, 0)),
                  pl.BlockSpec((1, 1, _BR), lambda i: (i, 0, 0)),
                  pl.BlockSpec((_F, _F), lambda i: (0, 0)),
                  pl.BlockSpec((1, _F), lambda i: (0, 0)),
                  pl.BlockSpec((_F, _G), lambda i: (0, 0)),
                  pl.BlockSpec((1, _G), lambda i: (0, 0))],
        out_specs=pl.BlockSpec((_G, _G), lambda i: (0, 0)),
        out_shape=jax.ShapeDtypeStruct((_G, _G), jnp.float32),
        scratch_shapes=[pltpu.VMEM((_G, _F), jnp.float32),
                        pltpu.VMEM((_G, _F), jnp.float32)],
    )(a0, a1, hs, dinv, b, batch3d, wh1, bh1, wh2, bh2)


def kernel(x, edge_index, batch, W1, b1, W2, b2, Wh1, bh1, Wh2, bh2):
    pad = _EPAD - _E
    src2d = jnp.concatenate(
        [edge_index[0], jnp.zeros((pad,), jnp.int32)]).reshape(_NW * _C, _K)
    dst2d = jnp.concatenate(
        [edge_index[1], jnp.full((pad,), _N, jnp.int32)]).reshape(_NW * _C, _K)
    ones16 = jnp.ones((_K, 16), jnp.float32)
    zeros16 = jnp.zeros((_TBL, 16), jnp.float32)
    zerosF = jnp.zeros((_TBL, _F), jnp.float32)
    batch3d = batch.reshape(_N // _BR, 1, _BR)

    degs = _sc_deg(dst2d, ones16, zeros16)
    dinv = _tc_dinv(degs)[:_N]

    hs1 = _tc_mm(x, dinv, W1)
    acc1 = _sc_scatter(src2d, dst2d, hs1, zerosF)
    hs2 = _tc_mid(acc1[0, :_N], acc1[1, :_N], hs1, dinv,
                  b1.reshape(1, _F), W2)
    acc2 = _sc_scatter(src2d, dst2d, hs2, zerosF)
    return _tc_final(acc2[0, :_N], acc2[1, :_N], hs2, dinv,
                     b2.reshape(1, _F), batch3d,
                     Wh1, bh1.reshape(1, _F), Wh2, bh2.reshape(1, _G))


# speedup vs baseline: 5.8227x; 5.8227x over previous
import functools

import jax
import jax.numpy as jnp
from jax import lax
from jax.experimental import pallas as pl
from jax.experimental.pallas import tpu as pltpu
from jax.experimental.pallas import tpu_sc as plsc

_N = 10000
_E = 320000
_F = 128
_G = 64

_NC = 2
_NS = 16
_NW = _NC * _NS
_K = 128
_C = 80
_EPT = _C * _K
_EPAD = _NW * _EPT
_TBL = 10112
_RPT = _TBL // _NS

_mesh = plsc.VectorSubcoreMesh(core_axis_name="c", subcore_axis_name="s")


@functools.partial(
    pl.kernel,
    out_type=jax.ShapeDtypeStruct((_NC, _TBL, _F), jnp.float32),
    mesh=_mesh,
    scratch_types=[
        pltpu.VMEM((_K,), jnp.int32),
        pltpu.VMEM((_C, _K), jnp.int32),
        pltpu.VMEM((_K, _F), jnp.float32),
        pltpu.VMEM_SHARED((_TBL, _F), jnp.float32),
        pltpu.SemaphoreType.DMA,
    ],
)
def _sc_scatter(src_hbm, dst_hbm, hs_hbm, zeros_hbm, out_hbm,
                idx1, idst_v, rows_v, acc_sp, sem):
    c = lax.axis_index("c")
    s = lax.axis_index("s")
    wid = c * _NS + s
    pltpu.sync_copy(dst_hbm.at[pl.ds(wid * _C, _C)], idst_v)
    pltpu.sync_copy(zeros_hbm.at[pl.ds(s * _RPT, _RPT)],
                    acc_sp.at[pl.ds(s * _RPT, _RPT)])
    plsc.subcore_barrier()

    def body(j, carry):
        pltpu.sync_copy(src_hbm.at[pl.ds(wid * _EPT + j * _K, _K)], idx1)
        pltpu.async_copy(hs_hbm.at[idx1], rows_v, sem).wait()
        pltpu.sync_copy(rows_v, acc_sp.at[idst_v.at[j]], add=True)
        return carry

    lax.fori_loop(0, _C, body, 0)
    plsc.subcore_barrier()
    pltpu.sync_copy(acc_sp.at[pl.ds(s * _RPT, _RPT)],
                    out_hbm.at[c, pl.ds(s * _RPT, _RPT)])


def _dinv_body(degs_ref, out_ref):
    d = degs_ref[0, :, 0:1] + degs_ref[1, :, 0:1] + 1.0
    out_ref[...] = jnp.broadcast_to(lax.rsqrt(d), (_TBL, _F))


def _tc_dinv(degs):
    return pl.pallas_call(
        _dinv_body,
        out_shape=jax.ShapeDtypeStruct((_TBL, _F), jnp.float32),
    )(degs)


_BR = 1000


def _mm_body(x_ref, dinv_ref, w_ref, out_ref):
    out_ref[...] = dinv_ref[...] * jnp.dot(
        x_ref[...], w_ref[...], preferred_element_type=jnp.float32)


def _tc_mm(x, dinv, w):
    return pl.pallas_call(
        _mm_body,
        grid=(_N // _BR,),
        in_specs=[
            pl.BlockSpec((_BR, _F), lambda i: (i, 0)),
            pl.BlockSpec((_BR, _F), lambda i: (i, 0)),
            pl.BlockSpec((_F, _F), lambda i: (0, 0)),
        ],
        out_specs=pl.BlockSpec((_BR, _F), lambda i: (i, 0)),
        out_shape=jax.ShapeDtypeStruct((_N, _F), jnp.float32),
    )(x, dinv, w)


def _mid_body(a0_ref, a1_ref, hs_ref, dinv_ref, b_ref, w_ref, out_ref):
    h = a0_ref[...] + a1_ref[...] + hs_ref[...]
    h = jnp.maximum(dinv_ref[...] * h + b_ref[...], 0.0)
    out_ref[...] = dinv_ref[...] * jnp.dot(
        h, w_ref[...], preferred_element_type=jnp.float32)


def _tc_mid(a0, a1, hs, dinv, b, w):
    blk = pl.BlockSpec((_BR, _F), lambda i: (i, 0))
    return pl.pallas_call(
        _mid_body,
        grid=(_N // _BR,),
        in_specs=[blk, blk, blk, blk,
                  pl.BlockSpec((1, _F), lambda i: (0, 0)),
                  pl.BlockSpec((_F, _F), lambda i: (0, 0))],
        out_specs=blk,
        out_shape=jax.ShapeDtypeStruct((_N, _F), jnp.float32),
    )(a0, a1, hs, dinv, b, w)


def _final_body(a0_ref, a1_ref, hs_ref, dinv_ref, b_ref, batch_ref,
                wh1_ref, bh1_ref, wh2_ref, bh2_ref, out_ref,
                sums_s, cnt_s):
    i = pl.program_id(0)

    @pl.when(i == 0)
    def _():
        sums_s[...] = jnp.zeros_like(sums_s)
        cnt_s[...] = jnp.zeros_like(cnt_s)

    h = a0_ref[...] + a1_ref[...] + hs_ref[...]
    h = jnp.maximum(dinv_ref[...] * h + b_ref[...], 0.0)
    b_row = batch_ref[0]
    gid = lax.broadcasted_iota(jnp.int32, (_G, _BR), 0)
    onehot = (gid == b_row).astype(jnp.float32)
    sums_s[...] += jnp.dot(onehot, h, preferred_element_type=jnp.float32)
    cnt = jnp.sum(onehot, axis=1, keepdims=True)
    cnt_s[...] += jnp.broadcast_to(cnt, (_G, _F))

    @pl.when(i == _N // _BR - 1)
    def _():
        g = sums_s[...] / jnp.maximum(cnt_s[...], 1.0)
        z = jnp.maximum(
            jnp.dot(g, wh1_ref[...], preferred_element_type=jnp.float32)
            + bh1_ref[...], 0.0)
        out_ref[...] = jnp.dot(
            z, wh2_ref[...], preferred_element_type=jnp.float32) + bh2_ref[...]


def _tc_final(a0, a1, hs, dinv, b, batch3d, wh1, bh1, wh2, bh2):
    blk = pl.BlockSpec((_BR, _F), lambda i: (i, 0))
    return pl.pallas_call(
        _final_body,
        grid=(_N // _BR,),
        in_specs=[blk, blk, blk, blk,
                  pl.BlockSpec((1, _F), lambda i: (0, 0)),
                  pl.BlockSpec((1, 1, _BR), lambda i: (i, 0, 0)),
                  pl.BlockSpec((_F, _F), lambda i: (0, 0)),
                  pl.BlockSpec((1, _F), lambda i: (0, 0)),
                  pl.BlockSpec((_F, _G), lambda i: (0, 0)),
                  pl.BlockSpec((1, _G), lambda i: (0, 0))],
        out_specs=pl.BlockSpec((_G, _G), lambda i: (0, 0)),
        out_shape=jax.ShapeDtypeStruct((_G, _G), jnp.float32),
        scratch_shapes=[pltpu.VMEM((_G, _F), jnp.float32),
                        pltpu.VMEM((_G, _F), jnp.float32)],
    )(a0, a1, hs, dinv, b, batch3d, wh1, bh1, wh2, bh2)


def kernel(x, edge_index, batch, W1, b1, W2, b2, Wh1, bh1, Wh2, bh2):
    pad = _EPAD - _E
    src2d = jnp.concatenate(
        [edge_index[0], jnp.zeros((pad,), jnp.int32)]).reshape(_NW * _C, _K)
    dst2d = jnp.concatenate(
        [edge_index[1], jnp.full((pad,), _N, jnp.int32)]).reshape(_NW * _C, _K)
    zerosF = jnp.zeros((_TBL, _F), jnp.float32)
    batch3d = batch.reshape(_N // _BR, 1, _BR)

    onesF = jnp.ones((_N, _F), jnp.float32)
    dacc = _sc_scatter(src2d.reshape(-1), dst2d, onesF, zerosF)
    degs = dacc[:, :, :16]
    dinv = _tc_dinv(degs)[:_N]

    hs1 = _tc_mm(x, dinv, W1)
    acc1 = _sc_scatter(src2d.reshape(-1), dst2d, hs1, zerosF)
    hs2 = _tc_mid(acc1[0, :_N], acc1[1, :_N], hs1, dinv,
                  b1.reshape(1, _F), W2)
    acc2 = _sc_scatter(src2d.reshape(-1), dst2d, hs2, zerosF)
    return _tc_final(acc2[0, :_N], acc2[1, :_N], hs2, dinv,
                     b2.reshape(1, _F), batch3d,
                     Wh1, bh1.reshape(1, _F), Wh2, bh2.reshape(1, _G))


# speedup vs baseline: 6.6654x; 1.1447x over previous
import functools

import jax
import jax.numpy as jnp
from jax import lax
from jax.experimental import pallas as pl
from jax.experimental.pallas import tpu as pltpu
from jax.experimental.pallas import tpu_sc as plsc

_N = 10000
_E = 320000
_F = 128
_G = 64

_NC = 2
_NS = 16
_NW = _NC * _NS
_K = 128
_C = 80
_EPT = _C * _K
_EPAD = _NW * _EPT
_TBL = 10112
_RPT = _TBL // _NS

_mesh = plsc.VectorSubcoreMesh(core_axis_name="c", subcore_axis_name="s")


def _make_scatter(width):
    @functools.partial(
        pl.kernel,
        out_type=jax.ShapeDtypeStruct((_NC, _TBL, width), jnp.float32),
        mesh=_mesh,
        scratch_types=[
            pltpu.VMEM((_K,), jnp.int32),
            pltpu.VMEM((_K,), jnp.int32),
            pltpu.VMEM((_C, _K), jnp.int32),
            pltpu.VMEM((_K, width), jnp.float32),
            pltpu.VMEM((_K, width), jnp.float32),
            pltpu.VMEM_SHARED((_TBL, width), jnp.float32),
            pltpu.SemaphoreType.DMA,
        ],
    )
    def _scat(src_hbm, dst_hbm, hs_hbm, zeros_hbm, out_hbm,
              idx1a, idx1b, idst_v, rows_a, rows_b, acc_sp, sem):
        c = lax.axis_index("c")
        s = lax.axis_index("s")
        wid = c * _NS + s
        base = wid * _EPT
        pltpu.sync_copy(dst_hbm.at[pl.ds(wid * _C, _C)], idst_v)
        pltpu.sync_copy(zeros_hbm.at[pl.ds(s * _RPT, _RPT)],
                        acc_sp.at[pl.ds(s * _RPT, _RPT)])
        plsc.subcore_barrier()

        pltpu.sync_copy(src_hbm.at[pl.ds(base, _K)], idx1a)
        pltpu.async_copy(hs_hbm.at[idx1a], rows_a, sem)

        def body(j2, carry):
            j = 2 * j2
            pltpu.sync_copy(src_hbm.at[pl.ds(base + (j + 1) * _K, _K)], idx1b)
            pltpu.make_async_copy(hs_hbm.at[idx1a], rows_a, sem).wait()
            pltpu.async_copy(hs_hbm.at[idx1b], rows_b, sem)
            pltpu.sync_copy(rows_a, acc_sp.at[idst_v.at[j]], add=True)

            @pl.when(j2 < _C // 2 - 1)
            def _():
                pltpu.sync_copy(src_hbm.at[pl.ds(base + (j + 2) * _K, _K)],
                                idx1a)
            pltpu.make_async_copy(hs_hbm.at[idx1b], rows_b, sem).wait()

            @pl.when(j2 < _C // 2 - 1)
            def _():
                pltpu.async_copy(hs_hbm.at[idx1a], rows_a, sem)
            pltpu.sync_copy(rows_b, acc_sp.at[idst_v.at[j + 1]], add=True)
            return carry

        lax.fori_loop(0, _C // 2, body, 0)
        plsc.subcore_barrier()
        pltpu.sync_copy(acc_sp.at[pl.ds(s * _RPT, _RPT)],
                        out_hbm.at[c, pl.ds(s * _RPT, _RPT)])

    return _scat


_sc_scatter = _make_scatter(_F)


def _dinv_body(degs_ref, out_ref):
    d = degs_ref[0, :, 0:1] + degs_ref[1, :, 0:1] + 1.0
    out_ref[...] = jnp.broadcast_to(lax.rsqrt(d), (_TBL, _F))


def _tc_dinv(degs):
    return pl.pallas_call(
        _dinv_body,
        out_shape=jax.ShapeDtypeStruct((_TBL, _F), jnp.float32),
    )(degs)


_BR = 1000


def _mm_body(x_ref, dinv_ref, w_ref, out_ref):
    out_ref[...] = dinv_ref[...] * jnp.dot(
        x_ref[...], w_ref[...], preferred_element_type=jnp.float32)


def _tc_mm(x, dinv, w):
    return pl.pallas_call(
        _mm_body,
        grid=(_N // _BR,),
        in_specs=[
            pl.BlockSpec((_BR, _F), lambda i: (i, 0)),
            pl.BlockSpec((_BR, _F), lambda i: (i, 0)),
            pl.BlockSpec((_F, _F), lambda i: (0, 0)),
        ],
        out_specs=pl.BlockSpec((_BR, _F), lambda i: (i, 0)),
        out_shape=jax.ShapeDtypeStruct((_N, _F), jnp.float32),
    )(x, dinv, w)


def _mid_body(a0_ref, a1_ref, hs_ref, dinv_ref, b_ref, w_ref, out_ref):
    h = a0_ref[...] + a1_ref[...] + hs_ref[...]
    h = jnp.maximum(dinv_ref[...] * h + b_ref[...], 0.0)
    out_ref[...] = dinv_ref[...] * jnp.dot(
        h, w_ref[...], preferred_element_type=jnp.float32)


def _tc_mid(a0, a1, hs, dinv, b, w):
    blk = pl.BlockSpec((_BR, _F), lambda i: (i, 0))
    return pl.pallas_call(
        _mid_body,
        grid=(_N // _BR,),
        in_specs=[blk, blk, blk, blk,
                  pl.BlockSpec((1, _F), lambda i: (0, 0)),
                  pl.BlockSpec((_F, _F), lambda i: (0, 0))],
        out_specs=blk,
        out_shape=jax.ShapeDtypeStruct((_N, _F), jnp.float32),
    )(a0, a1, hs, dinv, b, w)


def _final_body(a0_ref, a1_ref, hs_ref, dinv_ref, b_ref, batch_ref,
                wh1_ref, bh1_ref, wh2_ref, bh2_ref, out_ref,
                sums_s, cnt_s):
    i = pl.program_id(0)

    @pl.when(i == 0)
    def _():
        sums_s[...] = jnp.zeros_like(sums_s)
        cnt_s[...] = jnp.zeros_like(cnt_s)

    h = a0_ref[...] + a1_ref[...] + hs_ref[...]
    h = jnp.maximum(dinv_ref[...] * h + b_ref[...], 0.0)
    b_row = batch_ref[0]
    gid = lax.broadcasted_iota(jnp.int32, (_G, _BR), 0)
    onehot = (gid == b_row).astype(jnp.float32)
    sums_s[...] += jnp.dot(onehot, h, preferred_element_type=jnp.float32)
    cnt = jnp.sum(onehot, axis=1, keepdims=True)
    cnt_s[...] += jnp.broadcast_to(cnt, (_G, _F))

    @pl.when(i == _N // _BR - 1)
    def _():
        g = sums_s[...] / jnp.maximum(cnt_s[...], 1.0)
        z = jnp.maximum(
            jnp.dot(g, wh1_ref[...], preferred_element_type=jnp.float32)
            + bh1_ref[...], 0.0)
        out_ref[...] = jnp.dot(
            z, wh2_ref[...], preferred_element_type=jnp.float32) + bh2_ref[...]


def _tc_final(a0, a1, hs, dinv, b, batch3d, wh1, bh1, wh2, bh2):
    blk = pl.BlockSpec((_BR, _F), lambda i: (i, 0))
    return pl.pallas_call(
        _final_body,
        grid=(_N // _BR,),
        in_specs=[blk, blk, blk, blk,
                  pl.BlockSpec((1, _F), lambda i: (0, 0)),
                  pl.BlockSpec((1, 1, _BR), lambda i: (i, 0, 0)),
                  pl.BlockSpec((_F, _F), lambda i: (0, 0)),
                  pl.BlockSpec((1, _F), lambda i: (0, 0)),
                  pl.BlockSpec((_F, _G), lambda i: (0, 0)),
                  pl.BlockSpec((1, _G), lambda i: (0, 0))],
        out_specs=pl.BlockSpec((_G, _G), lambda i: (0, 0)),
        out_shape=jax.ShapeDtypeStruct((_G, _G), jnp.float32),
        scratch_shapes=[pltpu.VMEM((_G, _F), jnp.float32),
                        pltpu.VMEM((_G, _F), jnp.float32)],
    )(a0, a1, hs, dinv, b, batch3d, wh1, bh1, wh2, bh2)


def kernel(x, edge_index, batch, W1, b1, W2, b2, Wh1, bh1, Wh2, bh2):
    pad = _EPAD - _E
    src2d = jnp.concatenate(
        [edge_index[0], jnp.zeros((pad,), jnp.int32)]).reshape(_NW * _C, _K)
    dst2d = jnp.concatenate(
        [edge_index[1], jnp.full((pad,), _N, jnp.int32)]).reshape(_NW * _C, _K)
    zerosF = jnp.zeros((_TBL, _F), jnp.float32)
    batch3d = batch.reshape(_N // _BR, 1, _BR)

    onesF = jnp.ones((_N, _F), jnp.float32)
    dacc = _sc_scatter(src2d.reshape(-1), dst2d, onesF, zerosF)
    degs = dacc[:, :, :16]
    dinv = _tc_dinv(degs)[:_N]

    hs1 = _tc_mm(x, dinv, W1)
    acc1 = _sc_scatter(src2d.reshape(-1), dst2d, hs1, zerosF)
    hs2 = _tc_mid(acc1[0, :_N], acc1[1, :_N], hs1, dinv,
                  b1.reshape(1, _F), W2)
    acc2 = _sc_scatter(src2d.reshape(-1), dst2d, hs2, zerosF)
    return _tc_final(acc2[0, :_N], acc2[1, :_N], hs2, dinv,
                     b2.reshape(1, _F), batch3d,
                     Wh1, bh1.reshape(1, _F), Wh2, bh2.reshape(1, _G))
